# baseline (speedup 1.0000x reference)
import jax
import jax.numpy as jnp
from jax import lax
from jax.experimental import pallas as pl
from jax.experimental.pallas import tpu as pltpu
from jax.experimental.pallas import tpu_sc as plsc

N_NODES = 10000
D = 128
E = 320000

NC, NS, LANES = 2, 16, 16
NW = NC * NS
CH = 128
RPW = 80
EP = NW * RPW * CH
NCHUNK = EP // CH
ACC_ROWS = 10240
TPR = ACC_ROWS // NS
PAD_DST = 10048
NPAD_ROWS = 32

_SC_MESH = plsc.VectorSubcoreMesh(
    core_axis_name="c", subcore_axis_name="s", num_cores=NC, num_subcores=NS)


def _deg_body(col_hbm, deg_hbm, idx_v, ones_v, buf1_v, buf2_v, deg_sh, sem):
    c = lax.axis_index("c")
    s = lax.axis_index("s")
    wid = c * NS + s
    zero16 = jnp.zeros((LANES,), jnp.float32)
    one16 = jnp.ones((LANES,), jnp.float32)
    for k in range(TPR // LANES):
        buf1_v[pl.ds(k * LANES, LANES)] = zero16
    for k in range(CH // LANES):
        ones_v[pl.ds(k * LANES, LANES)] = one16
    pltpu.sync_copy(buf1_v, deg_sh.at[pl.ds(s * TPR, TPR)])
    pltpu.sync_copy(col_hbm.at[pl.ds(wid * RPW, RPW)], idx_v)
    plsc.subcore_barrier()

    def fire(j, carry):
        pltpu.async_copy(ones_v, deg_sh.at[idx_v.at[j]], sem, add=True)
        return carry

    lax.fori_loop(0, RPW, fire, 0)

    def drain(j, carry):
        pltpu.make_async_copy(ones_v, deg_sh.at[idx_v.at[j]], sem).wait()
        return carry

    lax.fori_loop(0, RPW, drain, 0)
    plsc.subcore_barrier()
    pltpu.sync_copy(deg_sh.at[pl.ds(s * TPR, TPR)], buf1_v)
    for k in range(TPR // LANES):
        buf2_v[k // 8, pl.ds((k % 8) * LANES, LANES)] = buf1_v[pl.ds(k * LANES, LANES)]
    pltpu.sync_copy(buf2_v, deg_hbm.at[c, s])


_deg_call = pl.kernel(
    _deg_body,
    out_type=jax.ShapeDtypeStruct((NC, NS, TPR // CH, CH), jnp.float32),
    mesh=_SC_MESH,
    scratch_types=[
        pltpu.VMEM((RPW, CH), jnp.int32),
        pltpu.VMEM((CH,), jnp.float32),
        pltpu.VMEM((TPR,), jnp.float32),
        pltpu.VMEM((TPR // CH, CH), jnp.float32),
        pltpu.VMEM_SHARED((ACC_ROWS,), jnp.float32),
        pltpu.SemaphoreType.DMA,
    ],
)


def _mm_body(x_ref, w_ref, dc_ref, y_ref, sc_ref):
    xw = jnp.dot(x_ref[...], w_ref[...], preferred_element_type=jnp.float32)
    scale = lax.rsqrt(dc_ref[...])
    y_ref[...] = xw * scale
    sc_ref[...] = scale


def _mm_call(x, w, deg_col):
    blk = 1000
    return pl.pallas_call(
        _mm_body,
        grid=(N_NODES // blk,),
        in_specs=[
            pl.BlockSpec((blk, D), lambda i: (i, 0)),
            pl.BlockSpec((D, D), lambda i: (0, 0)),
            pl.BlockSpec((blk, 1), lambda i: (i, 0)),
        ],
        out_specs=[
            pl.BlockSpec((blk, D), lambda i: (i, 0)),
            pl.BlockSpec((blk, 1), lambda i: (i, 0)),
        ],
        out_shape=[
            jax.ShapeDtypeStruct((N_NODES, D), jnp.float32),
            jax.ShapeDtypeStruct((N_NODES, 1), jnp.float32),
        ],
    )(x, w, deg_col)


NBUF = 2
HW = 64
RPP = 16
NPH = RPW // RPP


def _agg_body(y_hbm, row_hbm, col_hbm, acc_hbm,
              rids, cids, gbufs, acc_sh, gsems, ssems, isems):
    c = lax.axis_index("c")
    s = lax.axis_index("s")
    wid = c * NS + s
    base = wid * RPW
    zero16 = jnp.zeros((LANES,), jnp.float32)
    g0 = gbufs[0]

    def zrow(i, carry):
        for k in range(D // LANES):
            g0[i, pl.ds(k * LANES, LANES)] = zero16
        return carry

    lax.fori_loop(0, CH, zrow, 0)
    for k in range(TPR // CH):
        pltpu.sync_copy(g0, acc_sh.at[pl.ds(s * TPR + k * CH, CH)])
    plsc.subcore_barrier()

    def g_start(sl, j, b):
        pltpu.async_copy(y_hbm.at[rids[sl].at[j, pl.ds(0, HW)]],
                         gbufs[b].at[pl.ds(0, HW)], gsems[b])
        pltpu.async_copy(y_hbm.at[rids[sl].at[j, pl.ds(HW, HW)]],
                         gbufs[b].at[pl.ds(HW, HW)], gsems[b])

    def g_wait(sl, j, b):
        pltpu.make_async_copy(y_hbm.at[rids[sl].at[j, pl.ds(0, HW)]],
                              gbufs[b].at[pl.ds(0, HW)], gsems[b]).wait()
        pltpu.make_async_copy(y_hbm.at[rids[sl].at[j, pl.ds(HW, HW)]],
                              gbufs[b].at[pl.ds(HW, HW)], gsems[b]).wait()

    def s_start(sl, j, b):
        pltpu.async_copy(gbufs[b], acc_sh.at[cids[sl].at[j]], ssems[b], add=True)

    def s_wait(sl, j, b):
        pltpu.make_async_copy(gbufs[b], acc_sh.at[cids[sl].at[j]], ssems[b]).wait()

    def i_start(ph, sl):
        pltpu.async_copy(row_hbm.at[pl.ds(base + ph * RPP, RPP)], rids[sl],
                         isems[sl])
        pltpu.async_copy(col_hbm.at[pl.ds(base + ph * RPP, RPP)], cids[sl],
                         isems[sl])

    def i_wait(ph, sl):
        pltpu.make_async_copy(row_hbm.at[pl.ds(base + ph * RPP, RPP)],
                              rids[sl], isems[sl]).wait()
        pltpu.make_async_copy(col_hbm.at[pl.ds(base + ph * RPP, RPP)],
                              cids[sl], isems[sl]).wait()

    pltpu.sync_copy(row_hbm.at[pl.ds(base, RPP)], rids[0])
    pltpu.sync_copy(col_hbm.at[pl.ds(base, RPP)], cids[0])
    for b in range(NBUF):
        g_start(0, b, b)

    for h in range(NPH):
        sl = h % 2
        nsl = (h + 1) % 2
        if h + 1 < NPH:
            i_start(h + 1, nsl)

        def step(t, carry):
            j0 = NBUF * t
            for b in range(NBUF):
                g_wait(sl, j0 + b, b)
                s_start(sl, j0 + b, b)
            for b in range(NBUF):
                s_wait(sl, j0 + b, b)
                g_start(sl, j0 + b + NBUF, b)
            return carry

        lax.fori_loop(0, RPP // NBUF - 1, step, 0)
        if h + 1 < NPH:
            i_wait(h + 1, nsl)
        jt = RPP - NBUF
        for b in range(NBUF):
            g_wait(sl, jt + b, b)
            s_start(sl, jt + b, b)
        for b in range(NBUF):
            s_wait(sl, jt + b, b)
            if h + 1 < NPH:
                g_start(nsl, b, b)
    plsc.subcore_barrier()
    pltpu.sync_copy(acc_sh.at[pl.ds(s * TPR, TPR)],
                    acc_hbm.at[c, pl.ds(s * TPR, TPR)])


_agg_call = pl.kernel(
    _agg_body,
    out_type=jax.ShapeDtypeStruct((NC, ACC_ROWS, D), jnp.float32),
    mesh=_SC_MESH,
    scratch_types=[
        [pltpu.VMEM((RPP, CH), jnp.int32) for _ in range(2)],
        [pltpu.VMEM((RPP, CH), jnp.int32) for _ in range(2)],
        [pltpu.VMEM((CH, D), jnp.float32) for _ in range(NBUF)],
        pltpu.VMEM_SHARED((ACC_ROWS, D), jnp.float32),
        [pltpu.SemaphoreType.DMA for _ in range(NBUF)],
        [pltpu.SemaphoreType.DMA for _ in range(NBUF)],
        [pltpu.SemaphoreType.DMA for _ in range(2)],
    ],
)


def _out_body(acc_ref, y_ref, sc_ref, b_ref, o_ref):
    t = sc_ref[...] * (acc_ref[0] + acc_ref[1] + y_ref[...]) + b_ref[...]
    h = jnp.maximum(t, 0.0)
    m = jnp.max(h, axis=1, keepdims=True)
    e = jnp.exp(h - m)
    o_ref[...] = e / jnp.sum(e, axis=1, keepdims=True)


def _out_call(accs, y, scale_col, b2):
    blk = 1000
    return pl.pallas_call(
        _out_body,
        grid=(N_NODES // blk,),
        in_specs=[
            pl.BlockSpec((NC, blk, D), lambda i: (0, i, 0)),
            pl.BlockSpec((blk, D), lambda i: (i, 0)),
            pl.BlockSpec((blk, 1), lambda i: (i, 0)),
            pl.BlockSpec((1, D), lambda i: (0, 0)),
        ],
        out_specs=pl.BlockSpec((blk, D), lambda i: (i, 0)),
        out_shape=jax.ShapeDtypeStruct((N_NODES, D), jnp.float32),
    )(accs, y, scale_col, b2)


def kernel(x, edge_index, W, b):
    row = edge_index[0]
    col = edge_index[1]
    npad = EP - E
    ar = jnp.arange(npad, dtype=jnp.int32)
    row_p = jnp.concatenate([row, (ar * 131) % N_NODES]).reshape(NCHUNK, CH)
    col_p = jnp.concatenate([col, PAD_DST + (ar % NPAD_ROWS)]).reshape(NCHUNK, CH)

    degs = _deg_call(col_p)
    deg_col = ((degs[0] + degs[1]).reshape(-1)[:N_NODES]
               .reshape(N_NODES, 1) + 1.0)
    y, scale_col = _mm_call(x, W, deg_col)
    accs = _agg_call(y, row_p, col_p)
    return _out_call(accs, y, scale_col, b.reshape(1, D))


# speedup vs baseline: 1.1658x; 1.1658x over previous
import jax
import jax.numpy as jnp
from jax import lax
from jax.experimental import pallas as pl
from jax.experimental.pallas import tpu as pltpu
from jax.experimental.pallas import tpu_sc as plsc

N_NODES = 10000
D = 128
E = 320000

NC, NS, LANES = 2, 16, 16
NW = NC * NS
CH = 128
RPW = 80
EP = NW * RPW * CH
NCHUNK = EP // CH
ACC_ROWS = 10240
TPR = ACC_ROWS // NS
PAD_DST = 10048
NPAD_ROWS = 32

_SC_MESH = plsc.VectorSubcoreMesh(
    core_axis_name="c", subcore_axis_name="s", num_cores=NC, num_subcores=NS)


def _deg_body(col_hbm, deg_hbm, idx_v, ones_v, buf1_v, buf2_v, deg_sh, sem):
    c = lax.axis_index("c")
    s = lax.axis_index("s")
    wid = c * NS + s
    zero16 = jnp.zeros((LANES,), jnp.float32)
    one16 = jnp.ones((LANES,), jnp.float32)
    for k in range(TPR // LANES):
        buf1_v[pl.ds(k * LANES, LANES)] = zero16
    for k in range(64 // LANES):
        ones_v[pl.ds(k * LANES, LANES)] = one16
    pltpu.sync_copy(buf1_v, deg_sh.at[pl.ds(s * TPR, TPR)])
    pltpu.sync_copy(col_hbm.at[pl.ds(wid * (RPW * 2), RPW * 2)], idx_v)
    plsc.subcore_barrier()

    def fire(j, carry):
        pltpu.async_copy(ones_v, deg_sh.at[idx_v.at[j]], sem, add=True)
        return carry

    lax.fori_loop(0, RPW * 2, fire, 0)

    def drain(j, carry):
        pltpu.make_async_copy(ones_v, deg_sh.at[idx_v.at[j]], sem).wait()
        return carry

    lax.fori_loop(0, RPW * 2, drain, 0)
    plsc.subcore_barrier()
    pltpu.sync_copy(deg_sh.at[pl.ds(s * TPR, TPR)], buf1_v)
    for k in range(TPR // LANES):
        buf2_v[k // 8, pl.ds((k % 8) * LANES, LANES)] = buf1_v[pl.ds(k * LANES, LANES)]
    pltpu.sync_copy(buf2_v, deg_hbm.at[c, s])


_deg_call = pl.kernel(
    _deg_body,
    out_type=jax.ShapeDtypeStruct((NC, NS, TPR // CH, CH), jnp.float32),
    mesh=_SC_MESH,
    scratch_types=[
        pltpu.VMEM((RPW * 2, 64), jnp.int32),
        pltpu.VMEM((64,), jnp.float32),
        pltpu.VMEM((TPR,), jnp.float32),
        pltpu.VMEM((TPR // CH, CH), jnp.float32),
        pltpu.VMEM_SHARED((ACC_ROWS,), jnp.float32),
        pltpu.SemaphoreType.DMA,
    ],
)


def _mm_body(x_ref, w_ref, dc_ref, y_ref, sc_ref):
    xw = jnp.dot(x_ref[...], w_ref[...], preferred_element_type=jnp.float32)
    scale = lax.rsqrt(dc_ref[...])
    y_ref[...] = xw * scale
    sc_ref[...] = scale


def _mm_call(x, w, deg_col):
    blk = 1000
    return pl.pallas_call(
        _mm_body,
        grid=(N_NODES // blk,),
        in_specs=[
            pl.BlockSpec((blk, D), lambda i: (i, 0)),
            pl.BlockSpec((D, D), lambda i: (0, 0)),
            pl.BlockSpec((blk, 1), lambda i: (i, 0)),
        ],
        out_specs=[
            pl.BlockSpec((blk, D), lambda i: (i, 0)),
            pl.BlockSpec((blk, 1), lambda i: (i, 0)),
        ],
        out_shape=[
            jax.ShapeDtypeStruct((N_NODES, D), jnp.float32),
            jax.ShapeDtypeStruct((N_NODES, 1), jnp.float32),
        ],
    )(x, w, deg_col)


NBUF = 4
CW = 64
CPW = RPW * (CH // CW)
HCPW = CPW // 5


NPH = CPW // HCPW


def _agg_body(y_hbm, row_hbm, col_hbm, acc_hbm,
              rids, cids, gbufs, acc_sh, gsems, ssems, isems):
    c = lax.axis_index("c")
    s = lax.axis_index("s")
    wid = c * NS + s
    base = wid * CPW
    zero16 = jnp.zeros((LANES,), jnp.float32)
    g0 = gbufs[0]

    def zrow(i, carry):
        for k in range(D // LANES):
            g0[i, pl.ds(k * LANES, LANES)] = zero16
        return carry

    lax.fori_loop(0, CW, zrow, 0)
    for k in range(TPR // CW):
        pltpu.sync_copy(g0, acc_sh.at[pl.ds(s * TPR + k * CW, CW)])
    plsc.subcore_barrier()

    def g_start(sl, j, b):
        pltpu.async_copy(y_hbm.at[rids[sl].at[j]], gbufs[b], gsems[b])

    def g_wait(sl, j, b):
        pltpu.make_async_copy(y_hbm.at[rids[sl].at[j]], gbufs[b], gsems[b]).wait()

    def s_start(sl, j, b):
        pltpu.async_copy(gbufs[b], acc_sh.at[cids[sl].at[j]], ssems[b], add=True)

    def s_wait(sl, j, b):
        pltpu.make_async_copy(gbufs[b], acc_sh.at[cids[sl].at[j]], ssems[b]).wait()

    def i_start(ph, sl):
        pltpu.async_copy(row_hbm.at[pl.ds(base + ph * HCPW, HCPW)], rids[sl],
                         isems[sl])
        pltpu.async_copy(col_hbm.at[pl.ds(base + ph * HCPW, HCPW)], cids[sl],
                         isems[sl])

    def i_wait(ph, sl):
        pltpu.make_async_copy(row_hbm.at[pl.ds(base + ph * HCPW, HCPW)],
                              rids[sl], isems[sl]).wait()
        pltpu.make_async_copy(col_hbm.at[pl.ds(base + ph * HCPW, HCPW)],
                              cids[sl], isems[sl]).wait()

    pltpu.sync_copy(row_hbm.at[pl.ds(base, HCPW)], rids[0])
    pltpu.sync_copy(col_hbm.at[pl.ds(base, HCPW)], cids[0])
    for b in range(NBUF):
        g_start(0, b, b)

    for h in range(NPH):
        sl = h % 2
        nsl = (h + 1) % 2
        if h + 1 < NPH:
            i_start(h + 1, nsl)

        def step(t, carry):
            j0 = NBUF * t
            for b in range(NBUF):
                g_wait(sl, j0 + b, b)
                s_start(sl, j0 + b, b)
            for b in range(NBUF):
                s_wait(sl, j0 + b, b)
                g_start(sl, j0 + b + NBUF, b)
            return carry

        lax.fori_loop(0, HCPW // NBUF - 1, step, 0)
        if h + 1 < NPH:
            i_wait(h + 1, nsl)
        jt = HCPW - NBUF
        for b in range(NBUF):
            g_wait(sl, jt + b, b)
            s_start(sl, jt + b, b)
        for b in range(NBUF):
            s_wait(sl, jt + b, b)
            if h + 1 < NPH:
                g_start(nsl, b, b)
    plsc.subcore_barrier()
    pltpu.sync_copy(acc_sh.at[pl.ds(s * TPR, TPR)],
                    acc_hbm.at[c, pl.ds(s * TPR, TPR)])


_agg_call = pl.kernel(
    _agg_body,
    out_type=jax.ShapeDtypeStruct((NC, ACC_ROWS, D), jnp.float32),
    mesh=_SC_MESH,
    scratch_types=[
        [pltpu.VMEM((HCPW, CW), jnp.int32) for _ in range(2)],
        [pltpu.VMEM((HCPW, CW), jnp.int32) for _ in range(2)],
        [pltpu.VMEM((CW, D), jnp.float32) for _ in range(NBUF)],
        pltpu.VMEM_SHARED((ACC_ROWS, D), jnp.float32),
        [pltpu.SemaphoreType.DMA for _ in range(NBUF)],
        [pltpu.SemaphoreType.DMA for _ in range(NBUF)],
        [pltpu.SemaphoreType.DMA for _ in range(2)],
    ],
)


def _out_body(acc_ref, y_ref, sc_ref, b_ref, o_ref):
    t = sc_ref[...] * (acc_ref[0] + acc_ref[1] + y_ref[...]) + b_ref[...]
    h = jnp.maximum(t, 0.0)
    m = jnp.max(h, axis=1, keepdims=True)
    e = jnp.exp(h - m)
    o_ref[...] = e / jnp.sum(e, axis=1, keepdims=True)


def _out_call(accs, y, scale_col, b2):
    blk = 1000
    return pl.pallas_call(
        _out_body,
        grid=(N_NODES // blk,),
        in_specs=[
            pl.BlockSpec((NC, blk, D), lambda i: (0, i, 0)),
            pl.BlockSpec((blk, D), lambda i: (i, 0)),
            pl.BlockSpec((blk, 1), lambda i: (i, 0)),
            pl.BlockSpec((1, D), lambda i: (0, 0)),
        ],
        out_specs=pl.BlockSpec((blk, D), lambda i: (i, 0)),
        out_shape=jax.ShapeDtypeStruct((N_NODES, D), jnp.float32),
    )(accs, y, scale_col, b2)


def kernel(x, edge_index, W, b):
    row = edge_index[0]
    col = edge_index[1]
    npad = EP - E
    ar = jnp.arange(npad, dtype=jnp.int32)
    row64 = jnp.concatenate([row, (ar * 131) % N_NODES]).reshape(EP // CW, CW)
    col64 = jnp.concatenate([col, PAD_DST + (ar % NPAD_ROWS)]).reshape(EP // CW, CW)

    degs = _deg_call(col64)
    deg_col = ((degs[0] + degs[1]).reshape(-1)[:N_NODES]
               .reshape(N_NODES, 1) + 1.0)
    y, scale_col = _mm_call(x, W, deg_col)
    accs = _agg_call(y, row64, col64)
    return _out_call(accs, y, scale_col, b.reshape(1, D))
